# baseline (speedup 1.0000x reference)
import dataclasses
import functools
import math

import jax
import jax.numpy as jnp
from jax import lax
from jax.experimental import pallas as pl
from jax.experimental.pallas import tpu as pltpu
from jax.experimental.pallas import tpu_sc as plsc

N = 10000
E = 320000
D = 128
G = 64
NW = 32
ROWS = 320
NPAD = NW * ROWS
NEGB = -1.0e30


def _tck_a_body(x_ref, k_ref, w_ref, b_ref, o_ref):
    xw = jnp.dot(x_ref[...], w_ref[...], preferred_element_type=jnp.float32)
    xw = jnp.maximum(xw + b_ref[...], 0.0)
    o_ref[...] = jnp.where(k_ref[...] > 0.0, xw, NEGB)


def _tck_a(X, keptf, W, b2):
    return pl.pallas_call(
        _tck_a_body,
        out_shape=jax.ShapeDtypeStruct((NPAD, D), jnp.float32),
    )(X, keptf, W, b2)


def _tck_b1_body(a_ref, x_ref, wa_ref, wx_ref, k_ref, p_ref, h_ref, y_ref):
    h = jnp.dot(a_ref[...], wa_ref[...], preferred_element_type=jnp.float32)
    h += jnp.dot(x_ref[...], wx_ref[...], preferred_element_type=jnp.float32)
    h = jnp.maximum(h, 0.0)
    h = jnp.where(k_ref[...] > 0.0, h, 0.0)
    h_ref[...] = h
    p = p_ref[...]
    pn = p / jnp.sqrt(jnp.sum(p * p))
    y_ref[...] = jnp.dot(h, pn.T, preferred_element_type=jnp.float32)


def _tck_b1(aggr, X, Wu_a, Wu_x, keptf, p2):
    return pl.pallas_call(
        _tck_b1_body,
        out_shape=(jax.ShapeDtypeStruct((NPAD, D), jnp.float32),
                   jax.ShapeDtypeStruct((NPAD, 1), jnp.float32)),
    )(aggr, X, Wu_a, Wu_x, keptf, p2)


def _tck_b2_body(k_next, y_ref, k_ref, kn_ref, tn_ref):
    y = y_ref[...]
    yk = jnp.where(k_ref[...] > 0.0, y, -jnp.inf)
    u = lax.bitcast_convert_type(yk, jnp.uint32)
    key = jnp.where(u >> 31 != 0, ~u, u | jnp.uint32(0x80000000))

    def step(i, t):
        cand = t | (jnp.uint32(1) << (jnp.uint32(31) - i.astype(jnp.uint32)))
        cnt = jnp.sum((key >= cand).astype(jnp.int32))
        return jnp.where(cnt >= k_next, cand, t)

    t = lax.fori_loop(0, 32, step, jnp.uint32(0))
    gt = key > t
    eq = key == t
    needed = (k_next - jnp.sum(gt.astype(jnp.int32))).astype(jnp.float32)

    eqf = eq.astype(jnp.float32)
    ri = lax.broadcasted_iota(jnp.int32, (128, 128), 0)
    ci = lax.broadcasted_iota(jnp.int32, (128, 128), 1)
    mf = (ri < ci).astype(jnp.float32)
    inrow = jnp.dot(eqf, mf, preferred_element_type=jnp.float32)
    rowsum = jnp.sum(eqf, axis=1, keepdims=True)
    r8 = lax.broadcasted_iota(jnp.int32, (80, 80), 0)
    c8 = lax.broadcasted_iota(jnp.int32, (80, 80), 1)
    lf = (r8 > c8).astype(jnp.float32)
    rowpref = jnp.dot(lf, rowsum, preferred_element_type=jnp.float32)
    prefix = inrow + rowpref
    kept_new = gt | (eq & (prefix < needed))
    kn_ref[...] = kept_new.astype(jnp.float32)
    tn_ref[...] = jnp.tanh(y)


def _tck_b2(y2, keptf2, k_next):
    return pl.pallas_call(
        functools.partial(_tck_b2_body, k_next),
        out_shape=(jax.ShapeDtypeStruct((80, 128), jnp.float32),
                   jax.ShapeDtypeStruct((80, 128), jnp.float32)),
    )(y2, keptf2)


def _tck_b3_body(h_ref, kn_ref, tn_ref, b_ref, xn_ref, sm_ref, cnt_ref):
    xn = jnp.where(kn_ref[...] > 0.0, h_ref[...] * tn_ref[...], 0.0)
    xn_ref[...] = xn
    lanes = lax.broadcasted_iota(jnp.int32, (NPAD, 128), 1)
    onehot = (b_ref[...] == lanes).astype(jnp.float32)
    dn = (((0,), (0,)), ((), ()))
    sm_ref[...] = lax.dot_general(onehot, xn, dn,
                                  preferred_element_type=jnp.float32)
    cnt_ref[...] = lax.dot_general(onehot, kn_ref[...], dn,
                                   preferred_element_type=jnp.float32)


def _tck_b3(h, kn, tn, batch2d):
    return pl.pallas_call(
        _tck_b3_body,
        out_shape=(jax.ShapeDtypeStruct((NPAD, D), jnp.float32),
                   jax.ShapeDtypeStruct((128, D), jnp.float32),
                   jax.ShapeDtypeStruct((128, 1), jnp.float32)),
    )(h, kn, tn, batch2d)


def _tck_mlp_body(mx1_ref, mx2_ref, mx3_ref, sm1_ref, sm2_ref, sm3_ref,
                  c1_ref, c2_ref, c3_ref, w1_ref, b1_ref, w2_ref, b2_ref,
                  w3_ref, b3_ref, o_ref):
    def read(mx_ref, sm_ref, c_ref):
        mx = jnp.max(mx_ref[...], axis=0)
        mx = jnp.where(mx > -1.0e29, mx, 0.0)
        mean = sm_ref[...][:G] / jnp.maximum(c_ref[...][:G], 1.0)
        return jnp.concatenate([mx, mean], axis=1)

    z = (read(mx1_ref, sm1_ref, c1_ref) + read(mx2_ref, sm2_ref, c2_ref)
         + read(mx3_ref, sm3_ref, c3_ref))
    z = jnp.maximum(jnp.dot(z, w1_ref[...], preferred_element_type=jnp.float32)
                    + b1_ref[...], 0.0)
    z = jnp.maximum(jnp.dot(z, w2_ref[...], preferred_element_type=jnp.float32)
                    + b2_ref[...], 0.0)
    z = jnp.dot(z, w3_ref[...], preferred_element_type=jnp.float32) + b3_ref[...]
    o_ref[...] = 1.0 / (1.0 + jnp.exp(-z))


def _tck_mlp(mx1, mx2, mx3, sm1, sm2, sm3, c1, c2, c3, W1, b1, W2, b2, W3, b3):
    return pl.pallas_call(
        _tck_mlp_body,
        out_shape=jax.ShapeDtypeStruct((G, 1), jnp.float32),
    )(mx1, mx2, mx3, sm1, sm2, sm3, c1, c2, c3, W1, b1, W2, b2, W3, b3)


CH = 4000
NCH = E // CH
GB = 256

_sc_mesh = plsc.VectorSubcoreMesh(core_axis_name="c", subcore_axis_name="s")

_sc_params = pltpu.CompilerParams()
if "needs_layout_passes" in pltpu.CompilerParams.__dataclass_fields__:
    _sc_params = dataclasses.replace(_sc_params, needs_layout_passes=False)


def _sck_body(xw_hbm, src_hbm, dst_hbm, out_hbm,
              acc, dbuf, sbuf, cidx, cdloc, rows, cnt_smem):
    wid = lax.axis_index("s") * 2 + lax.axis_index("c")
    base = wid * ROWS

    pltpu.sync_copy(xw_hbm.at[pl.ds(base, ROWS)], acc)

    @pl.loop(0, GB, step=16)
    def _(i):
        cidx[pl.ds(i, 16)] = jnp.zeros((16,), jnp.int32)

    cnt_smem[0] = 0

    def flush():
        cnt = cnt_smem[0]
        pltpu.sync_copy(xw_hbm.at[cidx], rows)

        @pl.loop(0, cnt)
        def _(e):
            dl = cdloc[pl.ds(e, 16)][0]
            for v in range(8):
                sl = pl.ds(v * 16, 16)
                acc[dl, sl] = jnp.maximum(acc[dl, sl], rows[e, sl])

        cnt_smem[0] = 0

    @pl.loop(0, NCH)
    def _(c):
        pltpu.sync_copy(dst_hbm.at[pl.ds(c * CH, CH)], dbuf)
        pltpu.sync_copy(src_hbm.at[pl.ds(c * CH, CH)], sbuf)

        @pl.loop(0, CH, step=16)
        def _(j):
            d16 = dbuf[pl.ds(j, 16)]
            m = (d16 >= base) & (d16 < base + ROWS)

            @pl.when(jnp.max(m.astype(jnp.int32)) > 0)
            def _():
                mi = m.astype(jnp.int32)
                csum = plsc.cumsum(mi)
                cnt = cnt_smem[0]
                pos = csum - mi + cnt
                s16 = sbuf[pl.ds(j, 16)]
                plsc.store_scatter(cidx, [pos], s16, mask=m)
                plsc.store_scatter(cdloc, [pos], d16 - base, mask=m)
                cnt_smem[0] = cnt + jnp.max(csum)

                @pl.when(cnt_smem[0] >= GB - 16)
                def _():
                    flush()

    flush()
    pltpu.sync_copy(acc, out_hbm.at[pl.ds(base, ROWS)])


@jax.jit
def _segmax(xw, srcv, dstv):
    f = pl.kernel(
        _sck_body,
        out_type=jax.ShapeDtypeStruct((NPAD, D), jnp.float32),
        mesh=_sc_mesh,
        scratch_types=[
            pltpu.VMEM((ROWS, D), jnp.float32),
            pltpu.VMEM((CH,), jnp.int32),
            pltpu.VMEM((CH,), jnp.int32),
            pltpu.VMEM((GB,), jnp.int32),
            pltpu.VMEM((GB + 16,), jnp.int32),
            pltpu.VMEM((GB, D), jnp.float32),
            pltpu.SMEM((8,), jnp.int32),
        ],
        compiler_params=_sc_params,
    )
    return f(xw, srcv, dstv)


def _readout_max(Xn, kn, batchv):
    m = jax.ops.segment_max(jnp.where(kn > 0.0, Xn, NEGB), batchv,
                            num_segments=G)
    return jnp.where(jnp.isfinite(m), m, NEGB)[None]


def kernel(x, edge_index, batch, W_lin1, b_lin1, W_upd1, p1, W_lin2, b_lin2,
           W_upd2, p2, W_lin3, b_lin3, W_upd3, p3, W1, b1, W2, b2, W3, b3):
    srcv = edge_index[0]
    dstv = edge_index[1]
    X = jnp.pad(x, ((0, NPAD - N), (0, 0)))
    batchp = jnp.pad(batch, (0, NPAD - N), constant_values=G)
    batch2d = batchp[:, None]
    keptf = jnp.pad(jnp.ones((N, 1), jnp.float32), ((0, NPAD - N), (0, 0)))

    layer_params = (
        (W_lin1, b_lin1, W_upd1, p1),
        (W_lin2, b_lin2, W_upd2, p2),
        (W_lin3, b_lin3, W_upd3, p3),
    )
    n_cur = N
    mxs, sms, cnts = [], [], []
    for (W, b, Wu, p) in layer_params:
        k_next = int(math.ceil(0.8 * n_cur))
        xw = _tck_a(X, keptf, W, b[None])
        aggr = _segmax(xw, srcv, dstv)
        h, y = _tck_b1(aggr, X, Wu[:D], Wu[D:], keptf, p[None])
        kn2, tn2 = _tck_b2(y.reshape(80, 128), keptf.reshape(80, 128), k_next)
        kn = kn2.reshape(NPAD, 1)
        tn = tn2.reshape(NPAD, 1)
        Xn, sm, cnt = _tck_b3(h, kn, tn, batch2d)
        mxs.append(_readout_max(Xn, kn, batchp))
        sms.append(sm)
        cnts.append(cnt)
        X, keptf, n_cur = Xn, kn, k_next

    z = _tck_mlp(mxs[0], mxs[1], mxs[2], sms[0], sms[1], sms[2],
                 cnts[0], cnts[1], cnts[2], W1, b1[None], W2, b2[None],
                 W3, b3[None])
    return z[:, 0]


# speedup vs baseline: 1.6422x; 1.6422x over previous
import dataclasses
import functools
import math

import jax
import jax.numpy as jnp
from jax import lax
from jax.experimental import pallas as pl
from jax.experimental.pallas import tpu as pltpu
from jax.experimental.pallas import tpu_sc as plsc

N = 10000
E = 320000
D = 128
G = 64
NW = 32
ROWS = 320
NPAD = NW * ROWS
NEGB = -1.0e30


def _tck_a_body(x_ref, k_ref, w_ref, b_ref, o_ref):
    xw = jnp.dot(x_ref[...], w_ref[...], preferred_element_type=jnp.float32)
    xw = jnp.maximum(xw + b_ref[...], 0.0)
    o_ref[...] = jnp.where(k_ref[...] > 0.0, xw, NEGB)


def _tck_a(X, keptf, W, b2):
    return pl.pallas_call(
        _tck_a_body,
        out_shape=jax.ShapeDtypeStruct((NPAD, D), jnp.float32),
    )(X, keptf, W, b2)


def _tck_b1_body(a_ref, x_ref, wa_ref, wx_ref, k_ref, p_ref, h_ref, y_ref):
    h = jnp.dot(a_ref[...], wa_ref[...], preferred_element_type=jnp.float32)
    h += jnp.dot(x_ref[...], wx_ref[...], preferred_element_type=jnp.float32)
    h = jnp.maximum(h, 0.0)
    h = jnp.where(k_ref[...] > 0.0, h, 0.0)
    h_ref[...] = h
    p = p_ref[...]
    pn = p / jnp.sqrt(jnp.sum(p * p))
    y_ref[...] = jnp.dot(h, pn.T, preferred_element_type=jnp.float32)


def _tck_b1(aggr, X, Wu_a, Wu_x, keptf, p2):
    return pl.pallas_call(
        _tck_b1_body,
        out_shape=(jax.ShapeDtypeStruct((NPAD, D), jnp.float32),
                   jax.ShapeDtypeStruct((NPAD, 1), jnp.float32)),
    )(aggr, X, Wu_a, Wu_x, keptf, p2)


def _tck_b2_body(k_next, y_ref, k_ref, kn_ref, tn_ref):
    y = y_ref[...]
    yk = jnp.where(k_ref[...] > 0.0, y, -jnp.inf)
    u = lax.bitcast_convert_type(yk, jnp.uint32)
    key = jnp.where(u >> 31 != 0, ~u, u | jnp.uint32(0x80000000))

    def step(i, t):
        cand = t | (jnp.uint32(1) << (jnp.uint32(31) - i.astype(jnp.uint32)))
        cnt = jnp.sum((key >= cand).astype(jnp.int32))
        return jnp.where(cnt >= k_next, cand, t)

    t = lax.fori_loop(0, 32, step, jnp.uint32(0))
    gt = key > t
    eq = key == t
    needed = (k_next - jnp.sum(gt.astype(jnp.int32))).astype(jnp.float32)

    eqf = eq.astype(jnp.float32)
    ri = lax.broadcasted_iota(jnp.int32, (128, 128), 0)
    ci = lax.broadcasted_iota(jnp.int32, (128, 128), 1)
    mf = (ri < ci).astype(jnp.float32)
    inrow = jnp.dot(eqf, mf, preferred_element_type=jnp.float32)
    rowsum = jnp.sum(eqf, axis=1, keepdims=True)
    r8 = lax.broadcasted_iota(jnp.int32, (80, 80), 0)
    c8 = lax.broadcasted_iota(jnp.int32, (80, 80), 1)
    lf = (r8 > c8).astype(jnp.float32)
    rowpref = jnp.dot(lf, rowsum, preferred_element_type=jnp.float32)
    prefix = inrow + rowpref
    kept_new = gt | (eq & (prefix < needed))
    kn_ref[...] = kept_new.astype(jnp.float32)
    tn_ref[...] = jnp.tanh(y)


def _tck_b2(y2, keptf2, k_next):
    return pl.pallas_call(
        functools.partial(_tck_b2_body, k_next),
        out_shape=(jax.ShapeDtypeStruct((80, 128), jnp.float32),
                   jax.ShapeDtypeStruct((80, 128), jnp.float32)),
    )(y2, keptf2)


def _tck_b3_body(h_ref, kn_ref, tn_ref, b_ref, xn_ref, sm_ref, cnt_ref):
    xn = jnp.where(kn_ref[...] > 0.0, h_ref[...] * tn_ref[...], 0.0)
    xn_ref[...] = xn
    lanes = lax.broadcasted_iota(jnp.int32, (NPAD, 128), 1)
    onehot = (b_ref[...] == lanes).astype(jnp.float32)
    dn = (((0,), (0,)), ((), ()))
    sm_ref[...] = lax.dot_general(onehot, xn, dn,
                                  preferred_element_type=jnp.float32)
    cnt_ref[...] = lax.dot_general(onehot, kn_ref[...], dn,
                                   preferred_element_type=jnp.float32)


def _tck_b3(h, kn, tn, batch2d):
    return pl.pallas_call(
        _tck_b3_body,
        out_shape=(jax.ShapeDtypeStruct((NPAD, D), jnp.float32),
                   jax.ShapeDtypeStruct((128, D), jnp.float32),
                   jax.ShapeDtypeStruct((128, 1), jnp.float32)),
    )(h, kn, tn, batch2d)


def _tck_mlp_body(mx1_ref, mx2_ref, mx3_ref, sm1_ref, sm2_ref, sm3_ref,
                  c1_ref, c2_ref, c3_ref, w1_ref, b1_ref, w2_ref, b2_ref,
                  w3_ref, b3_ref, o_ref):
    def read(mx_ref, sm_ref, c_ref):
        mx = jnp.max(mx_ref[...], axis=0)
        mx = jnp.where(mx > -1.0e29, mx, 0.0)
        mean = sm_ref[...][:G] / jnp.maximum(c_ref[...][:G], 1.0)
        return jnp.concatenate([mx, mean], axis=1)

    z = (read(mx1_ref, sm1_ref, c1_ref) + read(mx2_ref, sm2_ref, c2_ref)
         + read(mx3_ref, sm3_ref, c3_ref))
    z = jnp.maximum(jnp.dot(z, w1_ref[...], preferred_element_type=jnp.float32)
                    + b1_ref[...], 0.0)
    z = jnp.maximum(jnp.dot(z, w2_ref[...], preferred_element_type=jnp.float32)
                    + b2_ref[...], 0.0)
    z = jnp.dot(z, w3_ref[...], preferred_element_type=jnp.float32) + b3_ref[...]
    o_ref[...] = 1.0 / (1.0 + jnp.exp(-z))


def _tck_mlp(mx1, mx2, mx3, sm1, sm2, sm3, c1, c2, c3, W1, b1, W2, b2, W3, b3):
    return pl.pallas_call(
        _tck_mlp_body,
        out_shape=jax.ShapeDtypeStruct((G, 1), jnp.float32),
    )(mx1, mx2, mx3, sm1, sm2, sm3, c1, c2, c3, W1, b1, W2, b2, W3, b3)


CH = 4000
NCH = E // CH
GB = 256
EC = E + E // 16 + 2 * GB

_sc_mesh = plsc.VectorSubcoreMesh(core_axis_name="c", subcore_axis_name="s")

_sc_params = pltpu.CompilerParams()
if "needs_layout_passes" in pltpu.CompilerParams.__dataclass_fields__:
    _sc_params = dataclasses.replace(_sc_params, needs_layout_passes=False)


def _bck_body(src_hbm, dst_hbm, slist_hbm, dlist_hbm,
              dbuf, sbuf, wsrc, wdloc, cvec, smem):
    wid = lax.axis_index("s") * 2 + lax.axis_index("c")
    base = wid * ROWS
    ebase = wid * EC

    @pl.loop(0, GB, step=16)
    def _(i):
        wsrc[pl.ds(i, 16)] = jnp.full((16,), base, jnp.int32)
        wdloc[pl.ds(i, 16)] = jnp.zeros((16,), jnp.int32)

    smem[0] = 0
    smem[1] = 0

    def flush():
        woff = smem[1] * 16
        pltpu.sync_copy(wsrc, slist_hbm.at[pl.ds(ebase + woff, GB)])
        pltpu.sync_copy(wdloc, dlist_hbm.at[pl.ds(ebase + woff, GB)])
        smem[1] = smem[1] + (smem[0] + 15) // 16
        smem[0] = 0

    @pl.loop(0, NCH)
    def _(c):
        pltpu.sync_copy(dst_hbm.at[pl.ds(c * CH, CH)], dbuf)
        pltpu.sync_copy(src_hbm.at[pl.ds(c * CH, CH)], sbuf)

        @pl.loop(0, CH, step=16)
        def _(j):
            d16 = dbuf[pl.ds(j, 16)]
            m = (d16 >= base) & (d16 < base + ROWS)

            @pl.when(jnp.max(m.astype(jnp.int32)) > 0)
            def _():
                mi = m.astype(jnp.int32)
                csum = plsc.cumsum(mi)
                cb = smem[0]
                pos = csum - mi + cb
                s16 = sbuf[pl.ds(j, 16)]
                plsc.store_scatter(wsrc, [pos], s16, mask=m)
                plsc.store_scatter(wdloc, [pos], d16 - base, mask=m)
                smem[0] = cb + jnp.max(csum)

                @pl.when(smem[0] >= GB - 16)
                def _():
                    flush()

    flush()
    pltpu.sync_copy(wsrc, slist_hbm.at[pl.ds(ebase + smem[1] * 16, GB)])
    pltpu.sync_copy(wdloc, dlist_hbm.at[pl.ds(ebase + smem[1] * 16, GB)])
    cvec[pl.ds(0, 16)] = jnp.full((16,), smem[1] * 16, jnp.int32)
    pltpu.sync_copy(cvec, dlist_hbm.at[pl.ds(NW * EC + wid * 16, 16)])


@jax.jit
def _bucket(srcv, dstv):
    f = pl.kernel(
        _bck_body,
        out_type=(jax.ShapeDtypeStruct((NW * EC,), jnp.int32),
                  jax.ShapeDtypeStruct((NW * EC + NW * 16,), jnp.int32)),
        mesh=_sc_mesh,
        scratch_types=[
            pltpu.VMEM((CH,), jnp.int32),
            pltpu.VMEM((CH,), jnp.int32),
            pltpu.VMEM((GB,), jnp.int32),
            pltpu.VMEM((GB,), jnp.int32),
            pltpu.VMEM((16,), jnp.int32),
            pltpu.SMEM((8,), jnp.int32),
        ],
        compiler_params=_sc_params,
    )
    return f(srcv, dstv)


def _sxk_body(xw_hbm, slist_hbm, dlist_hbm, out_hbm,
              acc, cidx, dbuf, rows, cvec):
    wid = lax.axis_index("s") * 2 + lax.axis_index("c")
    base = wid * ROWS
    ebase = wid * EC

    pltpu.sync_copy(xw_hbm.at[pl.ds(base, ROWS)], acc)
    pltpu.sync_copy(dlist_hbm.at[pl.ds(NW * EC + wid * 16, 16)], cvec)
    cnt = cvec[pl.ds(0, 16)][0]
    nb = (cnt + (GB - 1)) // GB

    @pl.loop(0, nb)
    def _(b):
        off = ebase + b * GB
        pltpu.sync_copy(slist_hbm.at[pl.ds(off, GB)], cidx)
        pltpu.sync_copy(dlist_hbm.at[pl.ds(off, GB)], dbuf.at[pl.ds(0, GB)])
        pltpu.sync_copy(xw_hbm.at[cidx], rows)

        @pl.loop(0, GB)
        def _(e):
            dl = dbuf[pl.ds(e, 16)][0]
            for v in range(8):
                sl = pl.ds(v * 16, 16)
                acc[dl, sl] = jnp.maximum(acc[dl, sl], rows[e, sl])

    pltpu.sync_copy(acc, out_hbm.at[pl.ds(base, ROWS)])


@jax.jit
def _segmax(xw, slist, dlist):
    f = pl.kernel(
        _sxk_body,
        out_type=jax.ShapeDtypeStruct((NPAD, D), jnp.float32),
        mesh=_sc_mesh,
        scratch_types=[
            pltpu.VMEM((ROWS, D), jnp.float32),
            pltpu.VMEM((GB,), jnp.int32),
            pltpu.VMEM((GB + 16,), jnp.int32),
            pltpu.VMEM((GB, D), jnp.float32),
            pltpu.VMEM((16,), jnp.int32),
        ],
        compiler_params=_sc_params,
    )
    return f(xw, slist, dlist)


def _readout_max(Xn, kn, batchv):
    m = jax.ops.segment_max(jnp.where(kn > 0.0, Xn, NEGB), batchv,
                            num_segments=G)
    return jnp.where(jnp.isfinite(m), m, NEGB)[None]


def kernel(x, edge_index, batch, W_lin1, b_lin1, W_upd1, p1, W_lin2, b_lin2,
           W_upd2, p2, W_lin3, b_lin3, W_upd3, p3, W1, b1, W2, b2, W3, b3):
    srcv = edge_index[0].astype(jnp.int32)
    dstv = edge_index[1].astype(jnp.int32)
    slist, dlist = _bucket(srcv, dstv)
    X = jnp.pad(x, ((0, NPAD - N), (0, 0)))
    batchp = jnp.pad(batch, (0, NPAD - N), constant_values=G)
    batch2d = batchp[:, None]
    keptf = jnp.pad(jnp.ones((N, 1), jnp.float32), ((0, NPAD - N), (0, 0)))

    layer_params = (
        (W_lin1, b_lin1, W_upd1, p1),
        (W_lin2, b_lin2, W_upd2, p2),
        (W_lin3, b_lin3, W_upd3, p3),
    )
    n_cur = N
    mxs, sms, cnts = [], [], []
    for (W, b, Wu, p) in layer_params:
        k_next = int(math.ceil(0.8 * n_cur))
        xw = _tck_a(X, keptf, W, b[None])
        aggr = _segmax(xw, slist, dlist)
        h, y = _tck_b1(aggr, X, Wu[:D], Wu[D:], keptf, p[None])
        kn2, tn2 = _tck_b2(y.reshape(80, 128), keptf.reshape(80, 128), k_next)
        kn = kn2.reshape(NPAD, 1)
        tn = tn2.reshape(NPAD, 1)
        Xn, sm, cnt = _tck_b3(h, kn, tn, batch2d)
        mxs.append(_readout_max(Xn, kn, batchp))
        sms.append(sm)
        cnts.append(cnt)
        X, keptf, n_cur = Xn, kn, k_next

    z = _tck_mlp(mxs[0], mxs[1], mxs[2], sms[0], sms[1], sms[2],
                 cnts[0], cnts[1], cnts[2], W1, b1[None], W2, b2[None],
                 W3, b3[None])
    return z[:, 0]


# speedup vs baseline: 1.6630x; 1.0127x over previous
import dataclasses
import functools
import math

import jax
import jax.numpy as jnp
from jax import lax
from jax.experimental import pallas as pl
from jax.experimental.pallas import tpu as pltpu
from jax.experimental.pallas import tpu_sc as plsc

N = 10000
E = 320000
D = 128
G = 64
NW = 32
ROWS = 320
NPAD = NW * ROWS
NEGB = -1.0e30


def _tck_a_body(x_ref, k_ref, w_ref, b_ref, o_ref):
    xw = jnp.dot(x_ref[...], w_ref[...], preferred_element_type=jnp.float32)
    xw = jnp.maximum(xw + b_ref[...], 0.0)
    o_ref[...] = jnp.where(k_ref[...] > 0.0, xw, NEGB)


def _tck_a(X, keptf, W, b2):
    return pl.pallas_call(
        _tck_a_body,
        out_shape=jax.ShapeDtypeStruct((NPAD, D), jnp.float32),
    )(X, keptf, W, b2)


def _tck_b1_body(a_ref, x_ref, wa_ref, wx_ref, k_ref, p_ref, h_ref, y_ref):
    h = jnp.dot(a_ref[...], wa_ref[...], preferred_element_type=jnp.float32)
    h += jnp.dot(x_ref[...], wx_ref[...], preferred_element_type=jnp.float32)
    h = jnp.maximum(h, 0.0)
    h = jnp.where(k_ref[...] > 0.0, h, 0.0)
    h_ref[...] = h
    p = p_ref[...]
    pn = p / jnp.sqrt(jnp.sum(p * p))
    y_ref[...] = jnp.dot(h, pn.T, preferred_element_type=jnp.float32)


def _tck_b1(aggr, X, Wu_a, Wu_x, keptf, p2):
    return pl.pallas_call(
        _tck_b1_body,
        out_shape=(jax.ShapeDtypeStruct((NPAD, D), jnp.float32),
                   jax.ShapeDtypeStruct((NPAD, 1), jnp.float32)),
    )(aggr, X, Wu_a, Wu_x, keptf, p2)


def _tck_b2_body(k_next, y_ref, k_ref, kn_ref, tn_ref):
    y = y_ref[...]
    yk = jnp.where(k_ref[...] > 0.0, y, -jnp.inf)
    u = lax.bitcast_convert_type(yk, jnp.uint32)
    key = jnp.where(u >> 31 != 0, ~u, u | jnp.uint32(0x80000000))

    def step(i, t):
        cand = t | (jnp.uint32(1) << (jnp.uint32(31) - i.astype(jnp.uint32)))
        cnt = jnp.sum((key >= cand).astype(jnp.int32))
        return jnp.where(cnt >= k_next, cand, t)

    t = lax.fori_loop(0, 32, step, jnp.uint32(0))
    gt = key > t
    eq = key == t
    needed = (k_next - jnp.sum(gt.astype(jnp.int32))).astype(jnp.float32)

    eqf = eq.astype(jnp.float32)
    ri = lax.broadcasted_iota(jnp.int32, (128, 128), 0)
    ci = lax.broadcasted_iota(jnp.int32, (128, 128), 1)
    mf = (ri < ci).astype(jnp.float32)
    inrow = jnp.dot(eqf, mf, preferred_element_type=jnp.float32)
    rowsum = jnp.sum(eqf, axis=1, keepdims=True)
    r8 = lax.broadcasted_iota(jnp.int32, (80, 80), 0)
    c8 = lax.broadcasted_iota(jnp.int32, (80, 80), 1)
    lf = (r8 > c8).astype(jnp.float32)
    rowpref = jnp.dot(lf, rowsum, preferred_element_type=jnp.float32)
    prefix = inrow + rowpref
    kept_new = gt | (eq & (prefix < needed))
    kn_ref[...] = kept_new.astype(jnp.float32)
    tn_ref[...] = jnp.tanh(y)


def _tck_b2(y2, keptf2, k_next):
    return pl.pallas_call(
        functools.partial(_tck_b2_body, k_next),
        out_shape=(jax.ShapeDtypeStruct((80, 128), jnp.float32),
                   jax.ShapeDtypeStruct((80, 128), jnp.float32)),
    )(y2, keptf2)


def _tck_b3_body(h_ref, kn_ref, tn_ref, b_ref, xn_ref, sm_ref, cnt_ref):
    xn = jnp.where(kn_ref[...] > 0.0, h_ref[...] * tn_ref[...], 0.0)
    xn_ref[...] = xn
    lanes = lax.broadcasted_iota(jnp.int32, (NPAD, 128), 1)
    onehot = (b_ref[...] == lanes).astype(jnp.float32)
    dn = (((0,), (0,)), ((), ()))
    sm_ref[...] = lax.dot_general(onehot, xn, dn,
                                  preferred_element_type=jnp.float32)
    cnt_ref[...] = lax.dot_general(onehot, kn_ref[...], dn,
                                   preferred_element_type=jnp.float32)


def _tck_b3(h, kn, tn, batch2d):
    return pl.pallas_call(
        _tck_b3_body,
        out_shape=(jax.ShapeDtypeStruct((NPAD, D), jnp.float32),
                   jax.ShapeDtypeStruct((128, D), jnp.float32),
                   jax.ShapeDtypeStruct((128, 1), jnp.float32)),
    )(h, kn, tn, batch2d)


def _tck_mx_body(xn_ref, kn_ref, b_ref, o_ref):
    xnm = jnp.where(kn_ref[...] > 0.0, xn_ref[...], NEGB)
    b = b_ref[...]

    def body(g, carry):
        col = jnp.max(jnp.where(b == g, xnm, NEGB), axis=0, keepdims=True)
        o_ref[pl.ds(g, 1), :] = col
        return carry

    lax.fori_loop(0, G, body, 0)


def _tck_mx(xn, kn, batch2d):
    return pl.pallas_call(
        _tck_mx_body,
        out_shape=jax.ShapeDtypeStruct((G, D), jnp.float32),
    )(xn, kn, batch2d)


def _tck_mlp_body(mx1_ref, mx2_ref, mx3_ref, sm1_ref, sm2_ref, sm3_ref,
                  c1_ref, c2_ref, c3_ref, w1_ref, b1_ref, w2_ref, b2_ref,
                  w3_ref, b3_ref, o_ref):
    def read(mx_ref, sm_ref, c_ref):
        mx = mx_ref[...]
        mx = jnp.where(mx > -1.0e29, mx, 0.0)
        mean = sm_ref[...][:G] / jnp.maximum(c_ref[...][:G], 1.0)
        return jnp.concatenate([mx, mean], axis=1)

    z = (read(mx1_ref, sm1_ref, c1_ref) + read(mx2_ref, sm2_ref, c2_ref)
         + read(mx3_ref, sm3_ref, c3_ref))
    z = jnp.maximum(jnp.dot(z, w1_ref[...], preferred_element_type=jnp.float32)
                    + b1_ref[...], 0.0)
    z = jnp.maximum(jnp.dot(z, w2_ref[...], preferred_element_type=jnp.float32)
                    + b2_ref[...], 0.0)
    z = jnp.dot(z, w3_ref[...], preferred_element_type=jnp.float32) + b3_ref[...]
    o_ref[...] = 1.0 / (1.0 + jnp.exp(-z))


def _tck_mlp(mx1, mx2, mx3, sm1, sm2, sm3, c1, c2, c3, W1, b1, W2, b2, W3, b3):
    return pl.pallas_call(
        _tck_mlp_body,
        out_shape=jax.ShapeDtypeStruct((G, 1), jnp.float32),
    )(mx1, mx2, mx3, sm1, sm2, sm3, c1, c2, c3, W1, b1, W2, b2, W3, b3)


CH = 4000
NCH = E // CH
GB = 256
EC = E + E // 16 + 2 * GB

_sc_mesh = plsc.VectorSubcoreMesh(core_axis_name="c", subcore_axis_name="s")

_sc_params = pltpu.CompilerParams()
if "needs_layout_passes" in pltpu.CompilerParams.__dataclass_fields__:
    _sc_params = dataclasses.replace(_sc_params, needs_layout_passes=False)


def _bck_body(src_hbm, dst_hbm, slist_hbm, dlist_hbm,
              dbuf, sbuf, wsrc, wdloc, cvec, smem):
    wid = lax.axis_index("s") * 2 + lax.axis_index("c")
    base = wid * ROWS
    ebase = wid * EC

    @pl.loop(0, GB, step=16)
    def _(i):
        wsrc[pl.ds(i, 16)] = jnp.full((16,), base, jnp.int32)
        wdloc[pl.ds(i, 16)] = jnp.zeros((16,), jnp.int32)

    smem[0] = 0
    smem[1] = 0

    def flush():
        woff = smem[1] * 16
        pltpu.sync_copy(wsrc, slist_hbm.at[pl.ds(ebase + woff, GB)])
        pltpu.sync_copy(wdloc, dlist_hbm.at[pl.ds(ebase + woff, GB)])
        smem[1] = smem[1] + (smem[0] + 15) // 16
        smem[0] = 0

    @pl.loop(0, NCH)
    def _(c):
        pltpu.sync_copy(dst_hbm.at[pl.ds(c * CH, CH)], dbuf)
        pltpu.sync_copy(src_hbm.at[pl.ds(c * CH, CH)], sbuf)

        @pl.loop(0, CH, step=16)
        def _(j):
            d16 = dbuf[pl.ds(j, 16)]
            m = (d16 >= base) & (d16 < base + ROWS)

            @pl.when(jnp.max(m.astype(jnp.int32)) > 0)
            def _():
                mi = m.astype(jnp.int32)
                csum = plsc.cumsum(mi)
                cb = smem[0]
                pos = csum - mi + cb
                s16 = sbuf[pl.ds(j, 16)]
                plsc.store_scatter(wsrc, [pos], s16, mask=m)
                plsc.store_scatter(wdloc, [pos], d16 - base, mask=m)
                smem[0] = cb + jnp.max(csum)

                @pl.when(smem[0] >= GB - 16)
                def _():
                    flush()

    flush()
    pltpu.sync_copy(wsrc, slist_hbm.at[pl.ds(ebase + smem[1] * 16, GB)])
    pltpu.sync_copy(wdloc, dlist_hbm.at[pl.ds(ebase + smem[1] * 16, GB)])
    cvec[pl.ds(0, 16)] = jnp.full((16,), smem[1] * 16, jnp.int32)
    pltpu.sync_copy(cvec, dlist_hbm.at[pl.ds(NW * EC + wid * 16, 16)])


@jax.jit
def _bucket(srcv, dstv):
    f = pl.kernel(
        _bck_body,
        out_type=(jax.ShapeDtypeStruct((NW * EC,), jnp.int32),
                  jax.ShapeDtypeStruct((NW * EC + NW * 16,), jnp.int32)),
        mesh=_sc_mesh,
        scratch_types=[
            pltpu.VMEM((CH,), jnp.int32),
            pltpu.VMEM((CH,), jnp.int32),
            pltpu.VMEM((GB,), jnp.int32),
            pltpu.VMEM((GB,), jnp.int32),
            pltpu.VMEM((16,), jnp.int32),
            pltpu.SMEM((8,), jnp.int32),
        ],
        compiler_params=_sc_params,
    )
    return f(srcv, dstv)


def _sxk_body(xw_hbm, slist_hbm, dlist_hbm, out_hbm,
              acc, cidx, dbuf, rows, cvec):
    wid = lax.axis_index("s") * 2 + lax.axis_index("c")
    base = wid * ROWS
    ebase = wid * EC

    pltpu.sync_copy(xw_hbm.at[pl.ds(base, ROWS)], acc)
    pltpu.sync_copy(dlist_hbm.at[pl.ds(NW * EC + wid * 16, 16)], cvec)
    cnt = cvec[pl.ds(0, 16)][0]
    nb = (cnt + (GB - 1)) // GB

    @pl.loop(0, nb)
    def _(b):
        off = ebase + b * GB
        pltpu.sync_copy(slist_hbm.at[pl.ds(off, GB)], cidx)
        pltpu.sync_copy(dlist_hbm.at[pl.ds(off, GB)], dbuf.at[pl.ds(0, GB)])
        pltpu.sync_copy(xw_hbm.at[cidx], rows)

        @pl.loop(0, GB)
        def _(e):
            dl = dbuf[pl.ds(e, 16)][0]
            arow = acc.at[dl]
            rrow = rows.at[e]
            for v in range(8):
                sl = pl.ds(v * 16, 16)
                arow[sl] = jnp.maximum(arow[sl], rrow[sl])

    pltpu.sync_copy(acc, out_hbm.at[pl.ds(base, ROWS)])


@jax.jit
def _segmax(xw, slist, dlist):
    f = pl.kernel(
        _sxk_body,
        out_type=jax.ShapeDtypeStruct((NPAD, D), jnp.float32),
        mesh=_sc_mesh,
        scratch_types=[
            pltpu.VMEM((ROWS, D), jnp.float32),
            pltpu.VMEM((GB,), jnp.int32),
            pltpu.VMEM((GB + 16,), jnp.int32),
            pltpu.VMEM((GB, D), jnp.float32),
            pltpu.VMEM((16,), jnp.int32),
        ],
        compiler_params=_sc_params,
    )
    return f(xw, slist, dlist)


def kernel(x, edge_index, batch, W_lin1, b_lin1, W_upd1, p1, W_lin2, b_lin2,
           W_upd2, p2, W_lin3, b_lin3, W_upd3, p3, W1, b1, W2, b2, W3, b3):
    srcv = edge_index[0].astype(jnp.int32)
    dstv = edge_index[1].astype(jnp.int32)
    slist, dlist = _bucket(srcv, dstv)
    X = jnp.pad(x, ((0, NPAD - N), (0, 0)))
    batchp = jnp.pad(batch, (0, NPAD - N), constant_values=G)
    batch2d = batchp[:, None]
    keptf = jnp.pad(jnp.ones((N, 1), jnp.float32), ((0, NPAD - N), (0, 0)))

    layer_params = (
        (W_lin1, b_lin1, W_upd1, p1),
        (W_lin2, b_lin2, W_upd2, p2),
        (W_lin3, b_lin3, W_upd3, p3),
    )
    n_cur = N
    mxs, sms, cnts = [], [], []
    for (W, b, Wu, p) in layer_params:
        k_next = int(math.ceil(0.8 * n_cur))
        xw = _tck_a(X, keptf, W, b[None])
        aggr = _segmax(xw, slist, dlist)
        h, y = _tck_b1(aggr, X, Wu[:D], Wu[D:], keptf, p[None])
        kn2, tn2 = _tck_b2(y.reshape(80, 128), keptf.reshape(80, 128), k_next)
        kn = kn2.reshape(NPAD, 1)
        tn = tn2.reshape(NPAD, 1)
        Xn, sm, cnt = _tck_b3(h, kn, tn, batch2d)
        mxs.append(_tck_mx(Xn, kn, batch2d))
        sms.append(sm)
        cnts.append(cnt)
        X, keptf, n_cur = Xn, kn, k_next

    z = _tck_mlp(mxs[0], mxs[1], mxs[2], sms[0], sms[1], sms[2],
                 cnts[0], cnts[1], cnts[2], W1, b1[None], W2, b2[None],
                 W3, b3[None])
    return z[:, 0]


# speedup vs baseline: 1.9769x; 1.1887x over previous
import dataclasses
import functools
import math

import jax
import jax.numpy as jnp
from jax import lax
from jax.experimental import pallas as pl
from jax.experimental.pallas import tpu as pltpu
from jax.experimental.pallas import tpu_sc as plsc

N = 10000
E = 320000
D = 128
G = 64
NW = 32
ROWS = 320
NPAD = NW * ROWS
NEGB = -1.0e30


def _tck_a_body(x_ref, k_ref, w_ref, b_ref, o_ref):
    xw = jnp.dot(x_ref[...], w_ref[...], preferred_element_type=jnp.float32)
    xw = jnp.maximum(xw + b_ref[...], 0.0)
    o_ref[...] = jnp.where(k_ref[...] > 0.0, xw, NEGB)


def _tck_a(X, keptf, W, b2):
    return pl.pallas_call(
        _tck_a_body,
        out_shape=jax.ShapeDtypeStruct((NPAD, D), jnp.float32),
    )(X, keptf, W, b2)


def _tck_b1_body(a_ref, x_ref, wa_ref, wx_ref, k_ref, p_ref, h_ref, y_ref):
    A = a_ref[...]
    aggr = jnp.maximum(A[:NPAD], A[NPAD:])
    h = jnp.dot(aggr, wa_ref[...], preferred_element_type=jnp.float32)
    h += jnp.dot(x_ref[...], wx_ref[...], preferred_element_type=jnp.float32)
    h = jnp.maximum(h, 0.0)
    h = jnp.where(k_ref[...] > 0.0, h, 0.0)
    h_ref[...] = h
    p = p_ref[...]
    pn = p / jnp.sqrt(jnp.sum(p * p))
    y_ref[...] = jnp.dot(h, pn.T, preferred_element_type=jnp.float32)


def _tck_b1(aggr, X, Wu_a, Wu_x, keptf, p2):
    return pl.pallas_call(
        _tck_b1_body,
        out_shape=(jax.ShapeDtypeStruct((NPAD, D), jnp.float32),
                   jax.ShapeDtypeStruct((NPAD, 1), jnp.float32)),
    )(aggr, X, Wu_a, Wu_x, keptf, p2)


def _tck_b2_body(k_next, y_ref, k_ref, kn_ref, tn_ref):
    y = y_ref[...]
    yk = jnp.where(k_ref[...] > 0.0, y, -jnp.inf)
    u = lax.bitcast_convert_type(yk, jnp.uint32)
    key = jnp.where(u >> 31 != 0, ~u, u | jnp.uint32(0x80000000))

    def step(i, t):
        cand = t | (jnp.uint32(1) << (jnp.uint32(31) - i.astype(jnp.uint32)))
        cnt = jnp.sum((key >= cand).astype(jnp.int32))
        return jnp.where(cnt >= k_next, cand, t)

    t = lax.fori_loop(0, 32, step, jnp.uint32(0))
    gt = key > t
    eq = key == t
    needed = (k_next - jnp.sum(gt.astype(jnp.int32))).astype(jnp.float32)

    eqf = eq.astype(jnp.float32)
    ri = lax.broadcasted_iota(jnp.int32, (128, 128), 0)
    ci = lax.broadcasted_iota(jnp.int32, (128, 128), 1)
    mf = (ri < ci).astype(jnp.float32)
    inrow = jnp.dot(eqf, mf, preferred_element_type=jnp.float32)
    rowsum = jnp.sum(eqf, axis=1, keepdims=True)
    r8 = lax.broadcasted_iota(jnp.int32, (80, 80), 0)
    c8 = lax.broadcasted_iota(jnp.int32, (80, 80), 1)
    lf = (r8 > c8).astype(jnp.float32)
    rowpref = jnp.dot(lf, rowsum, preferred_element_type=jnp.float32)
    prefix = inrow + rowpref
    kept_new = gt | (eq & (prefix < needed))
    kn_ref[...] = kept_new.astype(jnp.float32)
    tn_ref[...] = jnp.tanh(y)


def _tck_b2(y2, keptf2, k_next):
    return pl.pallas_call(
        functools.partial(_tck_b2_body, k_next),
        out_shape=(jax.ShapeDtypeStruct((80, 128), jnp.float32),
                   jax.ShapeDtypeStruct((80, 128), jnp.float32)),
    )(y2, keptf2)


def _tck_b3_body(h_ref, kn_ref, tn_ref, b_ref, xn_ref, sm_ref, cnt_ref):
    xn = jnp.where(kn_ref[...] > 0.0, h_ref[...] * tn_ref[...], 0.0)
    xn_ref[...] = xn
    lanes = lax.broadcasted_iota(jnp.int32, (NPAD, 128), 1)
    onehot = (b_ref[...] == lanes).astype(jnp.float32)
    dn = (((0,), (0,)), ((), ()))
    sm_ref[...] = lax.dot_general(onehot, xn, dn,
                                  preferred_element_type=jnp.float32)
    cnt_ref[...] = lax.dot_general(onehot, kn_ref[...], dn,
                                   preferred_element_type=jnp.float32)


def _tck_b3(h, kn, tn, batch2d):
    return pl.pallas_call(
        _tck_b3_body,
        out_shape=(jax.ShapeDtypeStruct((NPAD, D), jnp.float32),
                   jax.ShapeDtypeStruct((128, D), jnp.float32),
                   jax.ShapeDtypeStruct((128, 1), jnp.float32)),
    )(h, kn, tn, batch2d)


def _tck_mx_body(xn_ref, kn_ref, b_ref, o_ref):
    xnm = jnp.where(kn_ref[...] > 0.0, xn_ref[...], NEGB)
    b = b_ref[...]

    def body(g, carry):
        col = jnp.max(jnp.where(b == g, xnm, NEGB), axis=0, keepdims=True)
        o_ref[pl.ds(g, 1), :] = col
        return carry

    lax.fori_loop(0, G, body, 0)


def _tck_mx(xn, kn, batch2d):
    return pl.pallas_call(
        _tck_mx_body,
        out_shape=jax.ShapeDtypeStruct((G, D), jnp.float32),
    )(xn, kn, batch2d)


def _tck_mlp_body(mx1_ref, mx2_ref, mx3_ref, sm1_ref, sm2_ref, sm3_ref,
                  c1_ref, c2_ref, c3_ref, w1_ref, b1_ref, w2_ref, b2_ref,
                  w3_ref, b3_ref, o_ref):
    def read(mx_ref, sm_ref, c_ref):
        mx = mx_ref[...]
        mx = jnp.where(mx > -1.0e29, mx, 0.0)
        mean = sm_ref[...][:G] / jnp.maximum(c_ref[...][:G], 1.0)
        return jnp.concatenate([mx, mean], axis=1)

    z = (read(mx1_ref, sm1_ref, c1_ref) + read(mx2_ref, sm2_ref, c2_ref)
         + read(mx3_ref, sm3_ref, c3_ref))
    z = jnp.maximum(jnp.dot(z, w1_ref[...], preferred_element_type=jnp.float32)
                    + b1_ref[...], 0.0)
    z = jnp.maximum(jnp.dot(z, w2_ref[...], preferred_element_type=jnp.float32)
                    + b2_ref[...], 0.0)
    z = jnp.dot(z, w3_ref[...], preferred_element_type=jnp.float32) + b3_ref[...]
    o_ref[...] = 1.0 / (1.0 + jnp.exp(-z))


def _tck_mlp(mx1, mx2, mx3, sm1, sm2, sm3, c1, c2, c3, W1, b1, W2, b2, W3, b3):
    return pl.pallas_call(
        _tck_mlp_body,
        out_shape=jax.ShapeDtypeStruct((G, 1), jnp.float32),
    )(mx1, mx2, mx3, sm1, sm2, sm3, c1, c2, c3, W1, b1, W2, b2, W3, b3)


CH = 4000
GB = 256
E2 = E // 2
NCH = E2 // CH
NR = 16
ROWS2 = NPAD // NR
EC = E2 + E2 // 16 + 2 * GB

_sc_mesh = plsc.VectorSubcoreMesh(core_axis_name="c", subcore_axis_name="s")

_sc_params = pltpu.CompilerParams()
if "needs_layout_passes" in pltpu.CompilerParams.__dataclass_fields__:
    _sc_params = dataclasses.replace(_sc_params, needs_layout_passes=False)


def _bck_body(src_hbm, dst_hbm, slist_hbm, dlist_hbm,
              dbuf, sbuf, wsrc, wdloc, cvec, smem):
    c = lax.axis_index("c")
    s = lax.axis_index("s")
    lid = s * 2 + c
    base = s * ROWS2
    ebase = lid * EC

    @pl.loop(0, GB, step=16)
    def _(i):
        wsrc[pl.ds(i, 16)] = jnp.full((16,), base, jnp.int32)
        wdloc[pl.ds(i, 16)] = jnp.zeros((16,), jnp.int32)

    smem[0] = 0
    smem[1] = 0

    def flush():
        woff = smem[1] * 16
        pltpu.sync_copy(wsrc, slist_hbm.at[pl.ds(ebase + woff, GB)])
        pltpu.sync_copy(wdloc, dlist_hbm.at[pl.ds(ebase + woff, GB)])
        smem[1] = smem[1] + (smem[0] + 15) // 16
        smem[0] = 0

    @pl.loop(0, NCH)
    def _(k):
        pltpu.sync_copy(dst_hbm.at[pl.ds(c * E2 + k * CH, CH)], dbuf)
        pltpu.sync_copy(src_hbm.at[pl.ds(c * E2 + k * CH, CH)], sbuf)

        @pl.loop(0, CH, step=16)
        def _(j):
            d16 = dbuf[pl.ds(j, 16)]
            m = (d16 >= base) & (d16 < base + ROWS2)

            @pl.when(jnp.max(m.astype(jnp.int32)) > 0)
            def _():
                mi = m.astype(jnp.int32)
                csum = plsc.cumsum(mi)
                cb = smem[0]
                pos = csum - mi + cb
                s16 = sbuf[pl.ds(j, 16)]
                plsc.store_scatter(wsrc, [pos], s16, mask=m)
                plsc.store_scatter(wdloc, [pos], d16 - base, mask=m)
                smem[0] = cb + jnp.max(csum)

                @pl.when(smem[0] >= GB - 16)
                def _():
                    flush()

    flush()
    pltpu.sync_copy(wsrc, slist_hbm.at[pl.ds(ebase + smem[1] * 16, GB)])
    pltpu.sync_copy(wdloc, dlist_hbm.at[pl.ds(ebase + smem[1] * 16, GB)])
    cvec[pl.ds(0, 16)] = jnp.full((16,), smem[1] * 16, jnp.int32)
    pltpu.sync_copy(cvec, dlist_hbm.at[pl.ds(NW * EC + lid * 16, 16)])


@jax.jit
def _bucket(srcv, dstv):
    f = pl.kernel(
        _bck_body,
        out_type=(jax.ShapeDtypeStruct((NW * EC,), jnp.int32),
                  jax.ShapeDtypeStruct((NW * EC + NW * 16,), jnp.int32)),
        mesh=_sc_mesh,
        scratch_types=[
            pltpu.VMEM((CH,), jnp.int32),
            pltpu.VMEM((CH,), jnp.int32),
            pltpu.VMEM((GB,), jnp.int32),
            pltpu.VMEM((GB,), jnp.int32),
            pltpu.VMEM((16,), jnp.int32),
            pltpu.SMEM((8,), jnp.int32),
        ],
        compiler_params=_sc_params,
    )
    return f(srcv, dstv)


def _sxk_body(xw_hbm, slist_hbm, dlist_hbm, out_hbm,
              acc, cidx, dbuf, rows, cvec):
    c = lax.axis_index("c")
    s = lax.axis_index("s")
    lid = s * 2 + c
    base = s * ROWS2
    ebase = lid * EC

    pltpu.sync_copy(xw_hbm.at[pl.ds(base, ROWS2)], acc)
    pltpu.sync_copy(dlist_hbm.at[pl.ds(NW * EC + lid * 16, 16)], cvec)
    cnt = cvec[pl.ds(0, 16)][0]
    nb = (cnt + (GB - 1)) // GB

    @pl.loop(0, nb)
    def _(b):
        off = ebase + b * GB
        pltpu.sync_copy(slist_hbm.at[pl.ds(off, GB)], cidx)
        pltpu.sync_copy(dlist_hbm.at[pl.ds(off, GB)], dbuf.at[pl.ds(0, GB)])
        pltpu.sync_copy(xw_hbm.at[cidx], rows)

        @pl.loop(0, GB)
        def _(e):
            dl = dbuf[pl.ds(e, 16)][0]
            arow = acc.at[dl]
            rrow = rows.at[e]
            for v in range(8):
                sl = pl.ds(v * 16, 16)
                arow[sl] = jnp.maximum(arow[sl], rrow[sl])

    pltpu.sync_copy(acc, out_hbm.at[pl.ds(c * NPAD + base, ROWS2)])


@jax.jit
def _segmax(xw, slist, dlist):
    f = pl.kernel(
        _sxk_body,
        out_type=jax.ShapeDtypeStruct((2 * NPAD, D), jnp.float32),
        mesh=_sc_mesh,
        scratch_types=[
            pltpu.VMEM((ROWS2, D), jnp.float32),
            pltpu.VMEM((GB,), jnp.int32),
            pltpu.VMEM((GB + 16,), jnp.int32),
            pltpu.VMEM((GB, D), jnp.float32),
            pltpu.VMEM((16,), jnp.int32),
        ],
        compiler_params=_sc_params,
    )
    return f(xw, slist, dlist)


def kernel(x, edge_index, batch, W_lin1, b_lin1, W_upd1, p1, W_lin2, b_lin2,
           W_upd2, p2, W_lin3, b_lin3, W_upd3, p3, W1, b1, W2, b2, W3, b3):
    srcv = edge_index[0].astype(jnp.int32)
    dstv = edge_index[1].astype(jnp.int32)
    slist, dlist = _bucket(srcv, dstv)
    X = jnp.pad(x, ((0, NPAD - N), (0, 0)))
    batchp = jnp.pad(batch, (0, NPAD - N), constant_values=G)
    batch2d = batchp[:, None]
    keptf = jnp.pad(jnp.ones((N, 1), jnp.float32), ((0, NPAD - N), (0, 0)))

    layer_params = (
        (W_lin1, b_lin1, W_upd1, p1),
        (W_lin2, b_lin2, W_upd2, p2),
        (W_lin3, b_lin3, W_upd3, p3),
    )
    n_cur = N
    mxs, sms, cnts = [], [], []
    for (W, b, Wu, p) in layer_params:
        k_next = int(math.ceil(0.8 * n_cur))
        xw = _tck_a(X, keptf, W, b[None])
        aggr = _segmax(xw, slist, dlist)
        h, y = _tck_b1(aggr, X, Wu[:D], Wu[D:], keptf, p[None])
        kn2, tn2 = _tck_b2(y.reshape(80, 128), keptf.reshape(80, 128), k_next)
        kn = kn2.reshape(NPAD, 1)
        tn = tn2.reshape(NPAD, 1)
        Xn, sm, cnt = _tck_b3(h, kn, tn, batch2d)
        mxs.append(_tck_mx(Xn, kn, batch2d))
        sms.append(sm)
        cnts.append(cnt)
        X, keptf, n_cur = Xn, kn, k_next

    z = _tck_mlp(mxs[0], mxs[1], mxs[2], sms[0], sms[1], sms[2],
                 cnts[0], cnts[1], cnts[2], W1, b1[None], W2, b2[None],
                 W3, b3[None])
    return z[:, 0]


# speedup vs baseline: 2.0925x; 1.0585x over previous
import dataclasses
import functools
import math

import jax
import jax.numpy as jnp
from jax import lax
from jax.experimental import pallas as pl
from jax.experimental.pallas import tpu as pltpu
from jax.experimental.pallas import tpu_sc as plsc

N = 10000
E = 320000
D = 128
G = 64
NW = 32
ROWS = 320
NPAD = NW * ROWS
NEGB = -1.0e30


def _tck_a_body(x_ref, k_ref, w_ref, b_ref, o_ref):
    xw = jnp.dot(x_ref[...], w_ref[...], preferred_element_type=jnp.float32)
    xw = jnp.maximum(xw + b_ref[...], 0.0)
    o_ref[...] = jnp.where(k_ref[...] > 0.0, xw, NEGB)


def _tck_a(X, keptf, W, b2):
    return pl.pallas_call(
        _tck_a_body,
        out_shape=jax.ShapeDtypeStruct((NPAD, D), jnp.float32),
    )(X, keptf, W, b2)


def _tck_b1_body(a_ref, x_ref, wa_ref, wx_ref, k_ref, p_ref, h_ref, y_ref):
    A = a_ref[...]
    aggr = jnp.maximum(A[:NPAD], A[NPAD:])
    h = jnp.dot(aggr, wa_ref[...], preferred_element_type=jnp.float32)
    h += jnp.dot(x_ref[...], wx_ref[...], preferred_element_type=jnp.float32)
    h = jnp.maximum(h, 0.0)
    h = jnp.where(k_ref[...] > 0.0, h, 0.0)
    h_ref[...] = h
    p = p_ref[...]
    pn = p / jnp.sqrt(jnp.sum(p * p))
    y_ref[...] = jnp.dot(h, pn.T, preferred_element_type=jnp.float32)


def _tck_b1(aggr, X, Wu_a, Wu_x, keptf, p2):
    return pl.pallas_call(
        _tck_b1_body,
        out_shape=(jax.ShapeDtypeStruct((NPAD, D), jnp.float32),
                   jax.ShapeDtypeStruct((NPAD, 1), jnp.float32)),
    )(aggr, X, Wu_a, Wu_x, keptf, p2)


def _tck_b2_body(k_next, y_ref, k_ref, kn_ref, tn_ref):
    y = y_ref[...]
    yk = jnp.where(k_ref[...] > 0.0, y, -jnp.inf)
    u = lax.bitcast_convert_type(yk, jnp.uint32)
    key = jnp.where(u >> 31 != 0, ~u, u | jnp.uint32(0x80000000))

    def step(i, t):
        cand = t | (jnp.uint32(1) << (jnp.uint32(31) - i.astype(jnp.uint32)))
        cnt = jnp.sum((key >= cand).astype(jnp.int32))
        return jnp.where(cnt >= k_next, cand, t)

    t = lax.fori_loop(0, 32, step, jnp.uint32(0))
    gt = key > t
    eq = key == t
    needed = (k_next - jnp.sum(gt.astype(jnp.int32))).astype(jnp.float32)

    eqf = eq.astype(jnp.float32)
    ri = lax.broadcasted_iota(jnp.int32, (128, 128), 0)
    ci = lax.broadcasted_iota(jnp.int32, (128, 128), 1)
    mf = (ri < ci).astype(jnp.float32)
    inrow = jnp.dot(eqf, mf, preferred_element_type=jnp.float32)
    rowsum = jnp.sum(eqf, axis=1, keepdims=True)
    r8 = lax.broadcasted_iota(jnp.int32, (80, 80), 0)
    c8 = lax.broadcasted_iota(jnp.int32, (80, 80), 1)
    lf = (r8 > c8).astype(jnp.float32)
    rowpref = jnp.dot(lf, rowsum, preferred_element_type=jnp.float32)
    prefix = inrow + rowpref
    kept_new = gt | (eq & (prefix < needed))
    kn_ref[...] = kept_new.astype(jnp.float32)
    tn_ref[...] = jnp.tanh(y)


def _tck_b2(y2, keptf2, k_next):
    return pl.pallas_call(
        functools.partial(_tck_b2_body, k_next),
        out_shape=(jax.ShapeDtypeStruct((80, 128), jnp.float32),
                   jax.ShapeDtypeStruct((80, 128), jnp.float32)),
    )(y2, keptf2)


def _tck_b3_body(h_ref, kn_ref, tn_ref, b_ref, xn_ref, sm_ref, cnt_ref):
    xn = jnp.where(kn_ref[...] > 0.0, h_ref[...] * tn_ref[...], 0.0)
    xn_ref[...] = xn
    lanes = lax.broadcasted_iota(jnp.int32, (NPAD, 128), 1)
    onehot = (b_ref[...] == lanes).astype(jnp.float32)
    dn = (((0,), (0,)), ((), ()))
    sm_ref[...] = lax.dot_general(onehot, xn, dn,
                                  preferred_element_type=jnp.float32)
    cnt_ref[...] = lax.dot_general(onehot, kn_ref[...], dn,
                                   preferred_element_type=jnp.float32)


def _tck_b3(h, kn, tn, batch2d):
    return pl.pallas_call(
        _tck_b3_body,
        out_shape=(jax.ShapeDtypeStruct((NPAD, D), jnp.float32),
                   jax.ShapeDtypeStruct((128, D), jnp.float32),
                   jax.ShapeDtypeStruct((128, 1), jnp.float32)),
    )(h, kn, tn, batch2d)


def _tck_mx_body(xn_ref, kn_ref, b_ref, o_ref):
    xnm = jnp.where(kn_ref[...] > 0.0, xn_ref[...], NEGB)
    b = b_ref[...]

    def body(g, carry):
        col = jnp.max(jnp.where(b == g, xnm, NEGB), axis=0, keepdims=True)
        o_ref[pl.ds(g, 1), :] = col
        return carry

    lax.fori_loop(0, G, body, 0)


def _tck_mx(xn, kn, batch2d):
    return pl.pallas_call(
        _tck_mx_body,
        out_shape=jax.ShapeDtypeStruct((G, D), jnp.float32),
    )(xn, kn, batch2d)


def _tck_mlp_body(mx1_ref, mx2_ref, mx3_ref, sm1_ref, sm2_ref, sm3_ref,
                  c1_ref, c2_ref, c3_ref, w1_ref, b1_ref, w2_ref, b2_ref,
                  w3_ref, b3_ref, o_ref):
    def read(mx_ref, sm_ref, c_ref):
        mx = mx_ref[...]
        mx = jnp.where(mx > -1.0e29, mx, 0.0)
        mean = sm_ref[...][:G] / jnp.maximum(c_ref[...][:G], 1.0)
        return jnp.concatenate([mx, mean], axis=1)

    z = (read(mx1_ref, sm1_ref, c1_ref) + read(mx2_ref, sm2_ref, c2_ref)
         + read(mx3_ref, sm3_ref, c3_ref))
    z = jnp.maximum(jnp.dot(z, w1_ref[...], preferred_element_type=jnp.float32)
                    + b1_ref[...], 0.0)
    z = jnp.maximum(jnp.dot(z, w2_ref[...], preferred_element_type=jnp.float32)
                    + b2_ref[...], 0.0)
    z = jnp.dot(z, w3_ref[...], preferred_element_type=jnp.float32) + b3_ref[...]
    o_ref[...] = 1.0 / (1.0 + jnp.exp(-z))


def _tck_mlp(mx1, mx2, mx3, sm1, sm2, sm3, c1, c2, c3, W1, b1, W2, b2, W3, b3):
    return pl.pallas_call(
        _tck_mlp_body,
        out_shape=jax.ShapeDtypeStruct((G, 1), jnp.float32),
    )(mx1, mx2, mx3, sm1, sm2, sm3, c1, c2, c3, W1, b1, W2, b2, W3, b3)


CH = 4000
GB = 256
GBR = 128
E2 = E // 2
NCH = E2 // CH
NR = 16
ROWS2 = NPAD // NR
EC = E2 + E2 // 16 + 2 * GB

_sc_mesh = plsc.VectorSubcoreMesh(core_axis_name="c", subcore_axis_name="s")

_sc_params = pltpu.CompilerParams()
if "needs_layout_passes" in pltpu.CompilerParams.__dataclass_fields__:
    _sc_params = dataclasses.replace(_sc_params, needs_layout_passes=False)


def _bck_body(src_hbm, dst_hbm, slist_hbm, dlist_hbm,
              dbuf, sbuf, wsrc, wdloc, cvec, smem):
    c = lax.axis_index("c")
    s = lax.axis_index("s")
    lid = s * 2 + c
    base = s * ROWS2
    ebase = lid * EC

    @pl.loop(0, GB, step=16)
    def _(i):
        wsrc[pl.ds(i, 16)] = jnp.full((16,), base, jnp.int32)
        wdloc[pl.ds(i, 16)] = jnp.zeros((16,), jnp.int32)

    smem[0] = 0
    smem[1] = 0

    def flush():
        woff = smem[1] * 16
        pltpu.sync_copy(wsrc, slist_hbm.at[pl.ds(ebase + woff, GB)])
        pltpu.sync_copy(wdloc, dlist_hbm.at[pl.ds(ebase + woff, GB)])
        smem[1] = smem[1] + (smem[0] + 15) // 16
        smem[0] = 0

    @pl.loop(0, NCH)
    def _(k):
        pltpu.sync_copy(dst_hbm.at[pl.ds(c * E2 + k * CH, CH)], dbuf)
        pltpu.sync_copy(src_hbm.at[pl.ds(c * E2 + k * CH, CH)], sbuf)

        @pl.loop(0, CH, step=16)
        def _(j):
            d16 = dbuf[pl.ds(j, 16)]
            m = (d16 >= base) & (d16 < base + ROWS2)

            @pl.when(jnp.max(m.astype(jnp.int32)) > 0)
            def _():
                mi = m.astype(jnp.int32)
                csum = plsc.cumsum(mi)
                cb = smem[0]
                pos = csum - mi + cb
                s16 = sbuf[pl.ds(j, 16)]
                plsc.store_scatter(wsrc, [pos], s16, mask=m)
                plsc.store_scatter(wdloc, [pos], d16 - base, mask=m)
                smem[0] = cb + jnp.max(csum)

                @pl.when(smem[0] >= GB - 16)
                def _():
                    flush()

    flush()
    pltpu.sync_copy(wsrc, slist_hbm.at[pl.ds(ebase + smem[1] * 16, GB)])
    pltpu.sync_copy(wdloc, dlist_hbm.at[pl.ds(ebase + smem[1] * 16, GB)])
    cvec[pl.ds(0, 16)] = jnp.full((16,), smem[1] * 16, jnp.int32)
    pltpu.sync_copy(cvec, dlist_hbm.at[pl.ds(NW * EC + lid * 16, 16)])


@jax.jit
def _bucket(srcv, dstv):
    f = pl.kernel(
        _bck_body,
        out_type=(jax.ShapeDtypeStruct((NW * EC,), jnp.int32),
                  jax.ShapeDtypeStruct((NW * EC + NW * 16,), jnp.int32)),
        mesh=_sc_mesh,
        scratch_types=[
            pltpu.VMEM((CH,), jnp.int32),
            pltpu.VMEM((CH,), jnp.int32),
            pltpu.VMEM((GB,), jnp.int32),
            pltpu.VMEM((GB,), jnp.int32),
            pltpu.VMEM((16,), jnp.int32),
            pltpu.SMEM((8,), jnp.int32),
        ],
        compiler_params=_sc_params,
    )
    return f(srcv, dstv)


def _sxk_body(xw_hbm, slist_hbm, dlist_hbm, out_hbm,
              acc, cidx0, cidx1, dbuf0, dbuf1, rows0, rows1, cvec,
              sem0, sem1):
    c = lax.axis_index("c")
    s = lax.axis_index("s")
    lid = s * 2 + c
    base = s * ROWS2
    ebase = lid * EC

    cidx = (cidx0, cidx1)
    dbuf = (dbuf0, dbuf1)
    rows = (rows0, rows1)
    sem = (sem0, sem1)

    pltpu.sync_copy(xw_hbm.at[pl.ds(base, ROWS2)], acc)
    pltpu.sync_copy(dlist_hbm.at[pl.ds(NW * EC + lid * 16, 16)], cvec)
    cnt = cvec[pl.ds(0, 16)][0]
    nb = (cnt + (GBR - 1)) // GBR

    def prefetch(b, p):
        off = ebase + b * GBR
        pltpu.sync_copy(slist_hbm.at[pl.ds(off, GBR)], cidx[p])
        pltpu.sync_copy(dlist_hbm.at[pl.ds(off, GBR)],
                        dbuf[p].at[pl.ds(0, GBR)])
        pltpu.async_copy(xw_hbm.at[cidx[p]], rows[p], sem[p])

    def stage(b, p):
        @pl.when(b + 1 < nb)
        def _():
            prefetch(b + 1, 1 - p)

        pltpu.make_async_copy(xw_hbm.at[cidx[p]], rows[p], sem[p]).wait()

        @pl.loop(0, GBR)
        def _(e):
            dl = dbuf[p][pl.ds(e, 16)][0]
            arow = acc.at[dl]
            rrow = rows[p].at[e]
            for v in range(8):
                sl = pl.ds(v * 16, 16)
                arow[sl] = jnp.maximum(arow[sl], rrow[sl])

    @pl.when(nb > 0)
    def _():
        prefetch(0, 0)

    @pl.loop(0, (nb + 1) // 2)
    def _(g):
        b0 = g * 2

        @pl.when(b0 < nb)
        def _():
            stage(b0, 0)

        @pl.when(b0 + 1 < nb)
        def _():
            stage(b0 + 1, 1)

    pltpu.sync_copy(acc, out_hbm.at[pl.ds(c * NPAD + base, ROWS2)])


@jax.jit
def _segmax(xw, slist, dlist):
    f = pl.kernel(
        _sxk_body,
        out_type=jax.ShapeDtypeStruct((2 * NPAD, D), jnp.float32),
        mesh=_sc_mesh,
        scratch_types=[
            pltpu.VMEM((ROWS2, D), jnp.float32),
            pltpu.VMEM((GBR,), jnp.int32),
            pltpu.VMEM((GBR,), jnp.int32),
            pltpu.VMEM((GBR + 16,), jnp.int32),
            pltpu.VMEM((GBR + 16,), jnp.int32),
            pltpu.VMEM((GBR, D), jnp.float32),
            pltpu.VMEM((GBR, D), jnp.float32),
            pltpu.VMEM((16,), jnp.int32),
            pltpu.SemaphoreType.DMA,
            pltpu.SemaphoreType.DMA,
        ],
        compiler_params=_sc_params,
    )
    return f(xw, slist, dlist)


def kernel(x, edge_index, batch, W_lin1, b_lin1, W_upd1, p1, W_lin2, b_lin2,
           W_upd2, p2, W_lin3, b_lin3, W_upd3, p3, W1, b1, W2, b2, W3, b3):
    srcv = edge_index[0].astype(jnp.int32)
    dstv = edge_index[1].astype(jnp.int32)
    slist, dlist = _bucket(srcv, dstv)
    X = jnp.pad(x, ((0, NPAD - N), (0, 0)))
    batchp = jnp.pad(batch, (0, NPAD - N), constant_values=G)
    batch2d = batchp[:, None]
    keptf = jnp.pad(jnp.ones((N, 1), jnp.float32), ((0, NPAD - N), (0, 0)))

    layer_params = (
        (W_lin1, b_lin1, W_upd1, p1),
        (W_lin2, b_lin2, W_upd2, p2),
        (W_lin3, b_lin3, W_upd3, p3),
    )
    n_cur = N
    mxs, sms, cnts = [], [], []
    for (W, b, Wu, p) in layer_params:
        k_next = int(math.ceil(0.8 * n_cur))
        xw = _tck_a(X, keptf, W, b[None])
        aggr = _segmax(xw, slist, dlist)
        h, y = _tck_b1(aggr, X, Wu[:D], Wu[D:], keptf, p[None])
        kn2, tn2 = _tck_b2(y.reshape(80, 128), keptf.reshape(80, 128), k_next)
        kn = kn2.reshape(NPAD, 1)
        tn = tn2.reshape(NPAD, 1)
        Xn, sm, cnt = _tck_b3(h, kn, tn, batch2d)
        mxs.append(_tck_mx(Xn, kn, batch2d))
        sms.append(sm)
        cnts.append(cnt)
        X, keptf, n_cur = Xn, kn, k_next

    z = _tck_mlp(mxs[0], mxs[1], mxs[2], sms[0], sms[1], sms[2],
                 cnts[0], cnts[1], cnts[2], W1, b1[None], W2, b2[None],
                 W3, b3[None])
    return z[:, 0]
